# baseline (speedup 1.0000x reference)
import functools

import jax
import jax.numpy as jnp
from jax import lax
from jax.experimental import pallas as pl
from jax.experimental.pallas import tpu as pltpu
from jax.experimental.pallas import tpu_sc as plsc

_B = 16384
_D = 64
_NPAIR = 50000


@functools.lru_cache(maxsize=None)
def _build():
    info = plsc.get_sparse_core_info()
    nc, ns = info.num_cores, info.num_subcores
    nw = nc * ns
    nj = _B // 128 // nw
    mesh = plsc.VectorSubcoreMesh(core_axis_name="c", subcore_axis_name="s")
    out_type = tuple(
        jax.ShapeDtypeStruct((8, 128, 8, 128), jnp.float32)
        for _ in range(6)
    )

    @functools.partial(
        pl.kernel,
        mesh=mesh,
        out_type=out_type,
        compiler_params=pltpu.CompilerParams(
            use_tc_tiling_on_sc=False, needs_layout_passes=False),
        scratch_types=[
            pltpu.VMEM((6, nj, 128), jnp.int32),
            pltpu.VMEM((6, nj, 128), jnp.int32),
            pltpu.VMEM((128, 128), jnp.float32),
            pltpu.VMEM((_D, 128), jnp.float32),
            pltpu.SemaphoreType.DMA,
        ],
    )
    def gather6(rp_u, rp_t, i_u, i_tp, i_tn, i_up, i_un, i_ta,
                o_u, o_tp, o_tn, o_up, o_un, o_ta,
                idx_v, pr_v, rows_v, outt_v, sem):
        wid = lax.axis_index("s") * nc + lax.axis_index("c")
        iota = lax.iota(jnp.int32, 16)
        tables = (rp_u, rp_t, rp_t, rp_u, rp_u, rp_t)
        idxs = (i_u, i_tp, i_tn, i_up, i_un, i_ta)
        outs = (o_u, o_tp, o_tn, o_up, o_un, o_ta)

        for gi in range(6):
            pltpu.sync_copy(idxs[gi].at[pl.ds(wid * nj, nj)], idx_v.at[gi])

        for gi in range(6):
            def chunk(cc, _, gi=gi):
                for m in range(8):
                    v = idx_v[gi, cc, pl.ds(16 * m, 16)]
                    pr_v[gi, cc, pl.ds(16 * m, 16)] = v >> 1
                pltpu.async_copy(tables[gi].at[pr_v.at[gi, cc]], rows_v,
                                 sem).wait()
                for h in range(8):
                    parv = (idx_v[gi, cc, pl.ds(16 * h, 16)] & 1) << 6
                    rowv = iota + 16 * h

                    def sel(k4, c2, parv=parv, rowv=rowv, h=h):
                        for dk in range(4):
                            k = 4 * k4 + dk
                            outt_v[k, pl.ds(16 * h, 16)] = plsc.load_gather(
                                rows_v, [rowv, parv + k])
                        return c2
                    lax.fori_loop(0, 16, sel, 0)
                c = wid * nj + cc
                for g in range(8):
                    pltpu.sync_copy(outt_v.at[pl.ds(8 * g, 8)],
                                    outs[gi].at[g, c])
                return _
            lax.fori_loop(0, nj, chunk, 0)

    return gather6


def kernel(x_user, x_track_pos, x_track_neg, x_user_pos, x_user_neg,
           x_track_anchor, users_vecs, tracks_vecs):
    gather6 = _build()
    idx2d = [
        x.reshape(_B // 128, 128)
        for x in (x_user, x_track_pos, x_track_neg, x_user_pos, x_user_neg,
                  x_track_anchor)
    ]
    rp_u = users_vecs.reshape(_NPAIR, 128)
    rp_t = tracks_vecs.reshape(_NPAIR, 128)
    outs = gather6(rp_u, rp_t, *idx2d)
    return tuple(
        o.transpose(1, 3, 0, 2).reshape(_B, _D) for o in outs
    )


# speedup vs baseline: 1.0016x; 1.0016x over previous
import functools

import jax
import jax.numpy as jnp
from jax import lax
from jax.experimental import pallas as pl
from jax.experimental.pallas import tpu as pltpu
from jax.experimental.pallas import tpu_sc as plsc

_B = 16384
_D = 64
_NPAIR = 50000


@functools.lru_cache(maxsize=None)
def _build():
    info = plsc.get_sparse_core_info()
    nc, ns = info.num_cores, info.num_subcores
    nw = nc * ns
    nj = _B // 128 // nw
    mesh = plsc.VectorSubcoreMesh(core_axis_name="c", subcore_axis_name="s")
    out_type = tuple(
        jax.ShapeDtypeStruct((8, 128, 8, 128), jnp.float32)
        for _ in range(6)
    )

    @functools.partial(
        pl.kernel,
        mesh=mesh,
        out_type=out_type,
        compiler_params=pltpu.CompilerParams(
            use_tc_tiling_on_sc=True, needs_layout_passes=False),
        scratch_types=[
            pltpu.VMEM((6, nj, 128), jnp.int32),
            pltpu.VMEM((6, nj, 128), jnp.int32),
            pltpu.VMEM((128, 128), jnp.float32),
            pltpu.VMEM((_D, 128), jnp.float32),
            pltpu.SemaphoreType.DMA,
        ],
    )
    def gather6(rp_u, rp_t, i_u, i_tp, i_tn, i_up, i_un, i_ta,
                o_u, o_tp, o_tn, o_up, o_un, o_ta,
                idx_v, pr_v, rows_v, outt_v, sem):
        wid = lax.axis_index("s") * nc + lax.axis_index("c")
        iota = lax.iota(jnp.int32, 16)
        tables = (rp_u, rp_t, rp_t, rp_u, rp_u, rp_t)
        idxs = (i_u, i_tp, i_tn, i_up, i_un, i_ta)
        outs = (o_u, o_tp, o_tn, o_up, o_un, o_ta)

        for gi in range(6):
            pltpu.sync_copy(idxs[gi].at[pl.ds(wid * nj, nj)], idx_v.at[gi])
        for gi in range(6):
            def mk_pr(cc, _, gi=gi):
                for m in range(8):
                    v = idx_v[gi, cc, pl.ds(16 * m, 16)]
                    pr_v[gi, cc, pl.ds(16 * m, 16)] = v >> 1
                return _
            lax.fori_loop(0, nj, mk_pr, 0)

        for gi in range(6):
            def chunk(cc, _, gi=gi):
                pltpu.async_copy(tables[gi].at[pr_v.at[gi, cc]], rows_v,
                                 sem).wait()
                parvs = [
                    (idx_v[gi, cc, pl.ds(16 * h, 16)] & 1) << 6
                    for h in range(8)
                ]
                rowvs = [iota + 16 * h for h in range(8)]

                def sel(k16, _2):
                    for dk in range(16):
                        k = 16 * k16 + dk
                        for h in range(8):
                            outt_v[k, pl.ds(16 * h, 16)] = (
                                plsc.load_gather(
                                    rows_v, [rowvs[h], parvs[h] + k]))
                    return _2
                lax.fori_loop(0, 4, sel, 0)
                c = wid * nj + cc
                for g in range(8):
                    pltpu.sync_copy(outt_v.at[pl.ds(8 * g, 8)],
                                    outs[gi].at[g, c])
                return _
            lax.fori_loop(0, nj, chunk, 0)

    return gather6


def kernel(x_user, x_track_pos, x_track_neg, x_user_pos, x_user_neg,
           x_track_anchor, users_vecs, tracks_vecs):
    gather6 = _build()
    idx2d = [
        x.reshape(_B // 128, 128)
        for x in (x_user, x_track_pos, x_track_neg, x_user_pos, x_user_neg,
                  x_track_anchor)
    ]
    rp_u = users_vecs.reshape(_NPAIR, 128)
    rp_t = tracks_vecs.reshape(_NPAIR, 128)
    outs = gather6(rp_u, rp_t, *idx2d)
    return tuple(
        o.transpose(1, 3, 0, 2).reshape(_B, _D) for o in outs
    )


# speedup vs baseline: 1.4683x; 1.4659x over previous
import functools

import jax
import jax.numpy as jnp
from jax import lax
from jax.experimental import pallas as pl
from jax.experimental.pallas import tpu as pltpu
from jax.experimental.pallas import tpu_sc as plsc

_B = 16384
_D = 64
_NPAIR = 50000


@functools.lru_cache(maxsize=None)
def _build():
    info = plsc.get_sparse_core_info()
    nc, ns = info.num_cores, info.num_subcores
    nw = nc * ns
    nj = _B // 128 // nw
    mesh = plsc.VectorSubcoreMesh(core_axis_name="c", subcore_axis_name="s")
    out_type = tuple(
        jax.ShapeDtypeStruct((8, 128, 8, 128), jnp.float32)
        for _ in range(6)
    )

    @functools.partial(
        pl.kernel,
        mesh=mesh,
        out_type=out_type,
        compiler_params=pltpu.CompilerParams(
            use_tc_tiling_on_sc=True, needs_layout_passes=False),
        scratch_types=[
            pltpu.VMEM((6, nj, 128), jnp.int32),
            pltpu.VMEM((6, nj, 128), jnp.int32),
            pltpu.VMEM((128, 128), jnp.float32),
            pltpu.VMEM((_D, 128), jnp.float32),
            pltpu.SemaphoreType.DMA,
        ],
    )
    def gather6(rp_u, rp_t, i_u, i_tp, i_tn, i_up, i_un, i_ta,
                o_u, o_tp, o_tn, o_up, o_un, o_ta,
                idx_v, pr_v, rows_v, outt_v, sem):
        wid = lax.axis_index("s") * nc + lax.axis_index("c")
        iota = lax.iota(jnp.int32, 16)
        tables = (rp_u, rp_t, rp_t, rp_u, rp_u, rp_t)
        idxs = (i_u, i_tp, i_tn, i_up, i_un, i_ta)
        outs = (o_u, o_tp, o_tn, o_up, o_un, o_ta)

        for gi in range(6):
            pltpu.sync_copy(idxs[gi].at[pl.ds(wid * nj, nj)], idx_v.at[gi])
        for gi in range(6):
            def mk_pr(cc, _, gi=gi):
                for m in range(8):
                    v = idx_v[gi, cc, pl.ds(16 * m, 16)]
                    pr_v[gi, cc, pl.ds(16 * m, 16)] = v >> 1
                return _
            lax.fori_loop(0, nj, mk_pr, 0)

        for gi in range(6):
            def chunk(cc, _, gi=gi):
                pltpu.async_copy(tables[gi].at[pr_v.at[gi, cc]], rows_v,
                                 sem).wait()
                parvs = [
                    (idx_v[gi, cc, pl.ds(16 * h, 16)] & 1) << 6
                    for h in range(8)
                ]
                rowvs = [iota + 16 * h for h in range(8)]

                def sel(o16, _2):
                    for do in range(16):
                        kv = (iota + (16 * o16 + do)) & 63
                        for h in range(8):
                            v = plsc.load_gather(
                                rows_v, [rowvs[h], parvs[h] + kv])
                            plsc.store_scatter(outt_v, [kv, rowvs[h]], v)
                    return _2
                lax.fori_loop(0, 4, sel, 0)
                c = wid * nj + cc
                for g in range(8):
                    pltpu.sync_copy(outt_v.at[pl.ds(8 * g, 8)],
                                    outs[gi].at[g, c])
                return _
            lax.fori_loop(0, nj, chunk, 0)

    return gather6


def kernel(x_user, x_track_pos, x_track_neg, x_user_pos, x_user_neg,
           x_track_anchor, users_vecs, tracks_vecs):
    gather6 = _build()
    idx2d = [
        x.reshape(_B // 128, 128)
        for x in (x_user, x_track_pos, x_track_neg, x_user_pos, x_user_neg,
                  x_track_anchor)
    ]
    rp_u = users_vecs.reshape(_NPAIR, 128)
    rp_t = tracks_vecs.reshape(_NPAIR, 128)
    outs = gather6(rp_u, rp_t, *idx2d)
    return tuple(
        o.transpose(1, 3, 0, 2).reshape(_B, _D) for o in outs
    )


# speedup vs baseline: 1.8921x; 1.2887x over previous
import functools

import jax
import jax.numpy as jnp
from jax import lax
from jax.experimental import pallas as pl
from jax.experimental.pallas import tpu as pltpu
from jax.experimental.pallas import tpu_sc as plsc

_B = 16384
_D = 64
_V = 100000


@functools.lru_cache(maxsize=None)
def _build():
    info = plsc.get_sparse_core_info()
    nc, ns = info.num_cores, info.num_subcores
    nw = nc * ns
    nj = _B // 128 // nw
    nt = 6 * nj
    mesh = plsc.VectorSubcoreMesh(core_axis_name="c", subcore_axis_name="s")
    out_type = tuple(
        jax.ShapeDtypeStruct((8, 128, 8, 128), jnp.float32)
        for _ in range(6)
    )

    @functools.partial(
        pl.kernel,
        mesh=mesh,
        out_type=out_type,
        compiler_params=pltpu.CompilerParams(
            use_tc_tiling_on_sc=False, needs_layout_passes=False),
        scratch_types=[
            pltpu.VMEM((6, nj, 128), jnp.int32),
            pltpu.VMEM((128, 128), jnp.float32),
            pltpu.VMEM((128, 128), jnp.float32),
            pltpu.VMEM((_D, 128), jnp.float32),
            pltpu.VMEM((_D, 128), jnp.float32),
            pltpu.SemaphoreType.DMA,
            pltpu.SemaphoreType.DMA,
            pltpu.SemaphoreType.DMA,
        ],
    )
    def gather6(pd_u, pd_t, i_u, i_tp, i_tn, i_up, i_un, i_ta,
                o_u, o_tp, o_tn, o_up, o_un, o_ta,
                idx_v, rows0, rows1, outt0, outt1, sem_g, sem_o0, sem_o1):
        wid = lax.axis_index("s") * nc + lax.axis_index("c")
        iota = lax.iota(jnp.int32, 16)
        tables = (pd_u, pd_t, pd_t, pd_u, pd_u, pd_t)
        idxs = (i_u, i_tp, i_tn, i_up, i_un, i_ta)
        outs = (o_u, o_tp, o_tn, o_up, o_un, o_ta)
        rows = (rows0, rows1)
        outts = (outt0, outt1)
        sems_o = (sem_o0, sem_o1)
        rowvs = [iota + 16 * h for h in range(8)]

        for gi in range(6):
            pltpu.sync_copy(idxs[gi].at[pl.ds(wid * nj, nj)], idx_v.at[gi])

        chunks = [(gi, cc) for gi in range(6) for cc in range(nj)]
        g_hs = {}
        o_hs = {}
        g_hs[0] = pltpu.async_copy(
            tables[0].at[idx_v.at[0, 0]], rows[0], sem_g)
        for t, (gi, cc) in enumerate(chunks):
            b = t & 1
            g_hs[t].wait()
            if t + 1 < nt:
                gi2, cc2 = chunks[t + 1]
                g_hs[t + 1] = pltpu.async_copy(
                    tables[gi2].at[idx_v.at[gi2, cc2]], rows[1 - b], sem_g)
            if t >= 2:
                for h in o_hs.pop(t - 2):
                    h.wait()
            src, dst = rows[b], outts[b]

            def tp(ob, _, src=src, dst=dst):
                kv = (iota + ob) & 63
                for h in range(8):
                    v = plsc.load_gather(src, [rowvs[h], kv])
                    plsc.store_scatter(dst, [kv, rowvs[h]], v)
                return _
            lax.fori_loop(0, 64, tp, 0)
            c = wid * nj + cc
            o_hs[t] = [
                pltpu.async_copy(outts[b].at[pl.ds(8 * g, 8)],
                                 outs[gi].at[g, c], sems_o[b])
                for g in range(8)
            ]
        for t in (nt - 2, nt - 1):
            for h in o_hs.pop(t):
                h.wait()

    return gather6


def kernel(x_user, x_track_pos, x_track_neg, x_user_pos, x_user_neg,
           x_track_anchor, users_vecs, tracks_vecs):
    gather6 = _build()
    idx2d = [
        x.reshape(_B // 128, 128)
        for x in (x_user, x_track_pos, x_track_neg, x_user_pos, x_user_neg,
                  x_track_anchor)
    ]
    pd_u = jnp.pad(users_vecs, ((0, 0), (0, 64)))
    pd_t = jnp.pad(tracks_vecs, ((0, 0), (0, 64)))
    outs = gather6(pd_u, pd_t, *idx2d)
    return tuple(
        o.transpose(1, 3, 0, 2).reshape(_B, _D) for o in outs
    )


# speedup vs baseline: 1.9548x; 1.0331x over previous
import functools

import jax
import jax.numpy as jnp
from jax import lax
from jax.experimental import pallas as pl
from jax.experimental.pallas import tpu as pltpu
from jax.experimental.pallas import tpu_sc as plsc

_B = 16384
_D = 64
_V = 100000


@functools.lru_cache(maxsize=None)
def _build():
    info = plsc.get_sparse_core_info()
    nc, ns = info.num_cores, info.num_subcores
    nw = nc * ns
    nj = _B // 128 // nw
    nt = 3 * nj
    mesh = plsc.VectorSubcoreMesh(core_axis_name="c", subcore_axis_name="s")
    out_type = tuple(
        jax.ShapeDtypeStruct((8, 128, 8, 128), jnp.float32)
        for _ in range(3)
    )

    @functools.partial(
        pl.kernel,
        mesh=mesh,
        out_type=out_type,
        compiler_params=pltpu.CompilerParams(
            use_tc_tiling_on_sc=False, needs_layout_passes=False),
        scratch_types=[
            pltpu.VMEM((3, nj, 128), jnp.int32),
            pltpu.VMEM((128, 128), jnp.float32),
            pltpu.VMEM((128, 128), jnp.float32),
            pltpu.VMEM((_D, 128), jnp.float32),
            pltpu.VMEM((_D, 128), jnp.float32),
            pltpu.SemaphoreType.DMA,
            pltpu.SemaphoreType.DMA,
            pltpu.SemaphoreType.DMA,
        ],
    )
    def gather3(tbl, i_a, i_b, i_c, o_a, o_b, o_c,
                idx_v, rows0, rows1, outt0, outt1, sem_g, sem_o0, sem_o1):
        wid = lax.axis_index("s") * nc + lax.axis_index("c")
        iota = lax.iota(jnp.int32, 16)
        outs = (o_a, o_b, o_c)
        rows = (rows0, rows1)
        outts = (outt0, outt1)
        sems_o = (sem_o0, sem_o1)
        rowvs = [iota + 16 * h for h in range(8)]

        for gi, iref in enumerate((i_a, i_b, i_c)):
            pltpu.sync_copy(iref.at[pl.ds(wid * nj, nj)], idx_v.at[gi])

        chunks = [(gi, cc) for gi in range(3) for cc in range(nj)]
        g_hs = {}
        o_hs = {}
        g_hs[0] = pltpu.async_copy(tbl.at[idx_v.at[0, 0]], rows[0], sem_g)
        for t, (gi, cc) in enumerate(chunks):
            b = t & 1
            g_hs[t].wait()
            if t + 1 < nt:
                gi2, cc2 = chunks[t + 1]
                g_hs[t + 1] = pltpu.async_copy(
                    tbl.at[idx_v.at[gi2, cc2]], rows[1 - b], sem_g)
            if t >= 2:
                for h in o_hs.pop(t - 2):
                    h.wait()
            src, dst = rows[b], outts[b]

            def tp(ob, _, src=src, dst=dst):
                kv = (iota + ob) & 63
                for h in range(8):
                    v = plsc.load_gather(src, [rowvs[h], kv])
                    plsc.store_scatter(dst, [kv, rowvs[h]], v)
                return _
            lax.fori_loop(0, 64, tp, 0)
            c = wid * nj + cc
            o_hs[t] = [
                pltpu.async_copy(outts[b].at[pl.ds(8 * g, 8)],
                                 outs[gi].at[g, c], sems_o[b])
                for g in range(8)
            ]
        for t in (nt - 2, nt - 1):
            for h in o_hs.pop(t):
                h.wait()

    return gather3


def kernel(x_user, x_track_pos, x_track_neg, x_user_pos, x_user_neg,
           x_track_anchor, users_vecs, tracks_vecs):
    gather3 = _build()

    def i2(x):
        return x.reshape(_B // 128, 128)

    pd_u = jnp.pad(users_vecs, ((0, 0), (0, 64)))
    pd_t = jnp.pad(tracks_vecs, ((0, 0), (0, 64)))
    u4, up4, un4 = gather3(pd_u, i2(x_user), i2(x_user_pos), i2(x_user_neg))
    tp4, tn4, ta4 = gather3(pd_t, i2(x_track_pos), i2(x_track_neg),
                            i2(x_track_anchor))

    def fin(o):
        return o.transpose(1, 3, 0, 2).reshape(_B, _D)

    return (fin(u4), fin(tp4), fin(tn4), fin(up4), fin(un4), fin(ta4))
